# baseline (speedup 1.0000x reference)
import jax
import jax.numpy as jnp
from jax import lax
from jax.experimental import pallas as pl
from jax.experimental.pallas import tpu as pltpu
from jax.experimental.pallas import tpu_sc as plsc

N = 100000
E = 3200000
IN_DIM = 128
HEADS = 4
REL_NUM = 64
NEG = 0.2

LANES = 16
NC = 2
NS = 16
NW = NC * NS
ROWS = E // 128
KM = 8
MACROS = ROWS // KM
NMAC = -(-MACROS // NW)
NPAD = 100096
NT = NPAD // NS
CW = 16

_BM = 2000


def _mlp_body(x_ref, w1_ref, b1_ref, w2_ref, b2_ref, rel_ref, watt_ref,
              batt_ref, h_ref, tab_ref):
    x = x_ref[...]
    z = jnp.maximum(
        jnp.dot(x, w1_ref[...], preferred_element_type=jnp.float32)
        + b1_ref[...], 0.0)
    h = (jnp.dot(z, w2_ref[...], preferred_element_type=jnp.float32)
         + b2_ref[...])
    bm = h.shape[0]
    h_ref[...] = jnp.concatenate(
        [jnp.ones((bm, 8), jnp.float32), h, jnp.zeros((bm, 4), jnp.float32)],
        axis=1)

    @pl.when(pl.program_id(0) == 0)
    def _():
        rel = rel_ref[...]
        e0 = (jnp.dot(rel, watt_ref[0], preferred_element_type=jnp.float32)
              + batt_ref[0])
        e1 = (jnp.dot(rel, watt_ref[1], preferred_element_type=jnp.float32)
              + batt_ref[1])
        e0 = jnp.exp(jnp.where(e0 >= 0, e0, NEG * e0))
        e1 = jnp.exp(jnp.where(e1 >= 0, e1, NEG * e1))
        zz = jnp.zeros((REL_NUM, 4), jnp.float32)
        tab_ref[0] = jnp.concatenate([e0, e1, e0, zz], axis=1)
        tab_ref[1] = jnp.concatenate([e0, e1, e1, zz], axis=1)


def _mlp(x, w1cat, b1cat, w2bd, b2row, rel_emb, watt, batt):
    grid = (N // _BM,)
    return pl.pallas_call(
        _mlp_body,
        grid=grid,
        in_specs=[
            pl.BlockSpec((_BM, IN_DIM), lambda i: (i, 0)),
            pl.BlockSpec((IN_DIM, 2 * IN_DIM), lambda i: (0, 0)),
            pl.BlockSpec((1, 2 * IN_DIM), lambda i: (0, 0)),
            pl.BlockSpec((2 * IN_DIM, HEADS), lambda i: (0, 0)),
            pl.BlockSpec((1, HEADS), lambda i: (0, 0)),
            pl.BlockSpec((REL_NUM, 16), lambda i: (0, 0)),
            pl.BlockSpec((2, 16, HEADS), lambda i: (0, 0, 0)),
            pl.BlockSpec((2, HEADS), lambda i: (0, 0)),
        ],
        out_specs=[
            pl.BlockSpec((_BM, CW), lambda i: (i, 0)),
            pl.BlockSpec((2, REL_NUM, CW), lambda i: (0, 0, 0)),
        ],
        out_shape=[
            jax.ShapeDtypeStruct((N, CW), jnp.float32),
            jax.ShapeDtypeStruct((2, REL_NUM, CW), jnp.float32),
        ],
    )(x, w1cat, b1cat, w2bd, b2row, rel_emb, watt, batt)


def _edge_pass_body(es_hbm, dst_hbm, h_hbm, tab_hbm, z_hbm,
                    acc_hbm, tabv, esv, dstv,
                    hrv0, hrv1, msgx0, msgx1, acc_sh,
                    gs0, gs1, ss0, ss1):
    c = lax.axis_index("c")
    s = lax.axis_index("s")

    pltpu.sync_copy(z_hbm.at[pl.ds(s * NT, NT)], acc_sh.at[pl.ds(s * NT, NT)])
    pltpu.sync_copy(tab_hbm, tabv)
    plsc.subcore_barrier()

    w = c * NS + s
    mstart = (w * MACROS) // NW
    mend = ((w + 1) * MACROS) // NW

    lane = lax.iota(jnp.int32, LANES)
    zeros16 = jnp.full((LANES,), 0, jnp.int32)
    bufs = [(hrv0, msgx0, gs0, ss0), (hrv1, msgx1, gs1, ss1)]

    def _gather(jr, buf, sem):
        return pltpu.async_copy(
            h_hbm.at[esv.at[pl.ds(jr * 256 + 128, 128)]], buf, sem)

    def macro(m, carry):
        mm = mstart + m

        @pl.when(mm < mend)
        def _():
            r0 = mm * KM
            pltpu.sync_copy(es_hbm.at[pl.ds(r0 * 256, KM * 256)], esv)
            pltpu.sync_copy(dst_hbm.at[pl.ds(r0, KM)], dstv)

            gd = [None] * KM
            sd = [None] * KM
            for j in (0, 1):
                gd[j] = _gather(j, bufs[j][0], bufs[j][2])
            for j in range(KM):
                hrv_b, msgx_b, gs_b, ss_b = bufs[j % 2]
                if j >= 2:
                    sd[j - 2].wait()
                gd[j].wait()
                base = j * 256

                def edge(e, ce, _hrv=hrv_b, _msgx=msgx_b, _base=base):
                    etb = plsc.load_gather(esv, [zeros16 + (_base + e)])
                    w16 = plsc.load_gather(tabv, [etb * CW + lane])
                    _msgx[e, :] = w16 * _hrv[e, :]
                    return ce
                lax.fori_loop(0, 128, edge, 0, unroll=8)

                sd[j] = pltpu.async_copy(
                    msgx_b, acc_sh.at[dstv.at[j]], ss_b, add=True)
                if j + 2 < KM:
                    gd[j + 2] = _gather(j + 2, hrv_b, gs_b)
            sd[KM - 2].wait()
            sd[KM - 1].wait()
        return carry

    lax.fori_loop(0, NMAC, macro, 0)
    plsc.subcore_barrier()

    pltpu.sync_copy(acc_sh.at[pl.ds(s * NT, NT)],
                    acc_hbm.at[c, pl.ds(s * NT, NT)])


def _edge_pass(es, dst, h, tab1d, z16):
    mesh = plsc.VectorSubcoreMesh(core_axis_name="c", subcore_axis_name="s",
                                  num_cores=NC, num_subcores=NS)
    scratch = [
        pltpu.VMEM((REL_NUM * CW,), jnp.float32),
        pltpu.VMEM((KM * 256,), jnp.int32),
        pltpu.VMEM((KM, 128), jnp.int32),
        pltpu.VMEM((128, CW), jnp.float32),
        pltpu.VMEM((128, CW), jnp.float32),
        pltpu.VMEM((128, CW), jnp.float32),
        pltpu.VMEM((128, CW), jnp.float32),
        pltpu.VMEM_SHARED((NPAD, CW), jnp.float32),
        pltpu.SemaphoreType.DMA,
        pltpu.SemaphoreType.DMA,
        pltpu.SemaphoreType.DMA,
        pltpu.SemaphoreType.DMA,
    ]
    f = pl.kernel(_edge_pass_body,
                  out_type=jax.ShapeDtypeStruct((NC, NPAD, CW), jnp.float32),
                  mesh=mesh, scratch_types=scratch,
                  compiler_params=pltpu.CompilerParams(
                      needs_layout_passes=False,
                      use_tc_tiling_on_sc=False))
    return f(es, dst, h, tab1d, z16)


def _edge_pass2_body(es_hbm, dst_hbm, hm_hbm, tab_hbm, z_hbm,
                     acc_hbm, tabv, esv, dstv,
                     hm0, hm1, ms0, ms1, acc_sh,
                     gs0, gs1, ss0, ss1):
    c = lax.axis_index("c")
    s = lax.axis_index("s")

    pltpu.sync_copy(z_hbm.at[pl.ds(s * NT, NT)], acc_sh.at[pl.ds(s * NT, NT)])
    pltpu.sync_copy(tab_hbm, tabv)
    plsc.subcore_barrier()

    w = c * NS + s
    mstart = (w * MACROS) // NW
    mend = ((w + 1) * MACROS) // NW

    lane = lax.iota(jnp.int32, LANES)
    l_d8 = lax.shift_right_logical(lane, 3)
    l_m8 = jnp.bitwise_and(lane, 7)
    l_m8c = jnp.bitwise_and(l_m8, 3)
    m_lo4 = l_m8 < 4
    zeros16 = jnp.full((LANES,), 0, jnp.int32)
    fzeros = jnp.full((LANES,), 0.0, jnp.float32)
    bufs = [(hm0, ms0, gs0, ss0), (hm1, ms1, gs1, ss1)]

    def _gather(jr, buf, sem):
        return pltpu.async_copy(
            hm_hbm.at[esv.at[pl.ds(jr * 256 + 128, 128)]], buf, sem)

    def macro(m, carry):
        mm = mstart + m

        @pl.when(mm < mend)
        def _():
            r0 = mm * KM
            pltpu.sync_copy(es_hbm.at[pl.ds(r0 * 256, KM * 256)], esv)
            pltpu.sync_copy(dst_hbm.at[pl.ds(r0, KM)], dstv)

            gd = [None] * KM
            sd = [None] * KM
            for j in (0, 1):
                gd[j] = _gather(j, bufs[j][0], bufs[j][2])
            for j in range(KM):
                hm_b, ms_b, gs_b, ss_b = bufs[j % 2]
                if j >= 2:
                    sd[j - 2].wait()
                gd[j].wait()
                base = j * 256

                def vec(v, cv, _hm=hm_b, _ms=ms_b, _base=base):
                    e2 = v * 2 + l_d8
                    etg = plsc.load_gather(esv, [zeros16 + _base + e2])
                    wg = plsc.load_gather(tabv, [etg * CW + 8 + l_m8c])
                    hmg = plsc.load_gather(_hm, [e2])
                    val = jnp.where(m_lo4, wg * hmg, fzeros)
                    plsc.store_scatter(_ms, [e2, l_m8], val)
                    return cv
                lax.fori_loop(0, 64, vec, 0, unroll=8)

                sd[j] = pltpu.async_copy(
                    ms_b, acc_sh.at[dstv.at[j]], ss_b, add=True)
                if j + 2 < KM:
                    gd[j + 2] = _gather(j + 2, hm_b, gs_b)
            sd[KM - 2].wait()
            sd[KM - 1].wait()
        return carry

    lax.fori_loop(0, NMAC, macro, 0)
    plsc.subcore_barrier()

    pltpu.sync_copy(acc_sh.at[pl.ds(s * NT, NT)],
                    acc_hbm.at[c, pl.ds(s * NT, NT)])


def _edge_pass2(es, dst, hm, tab1d, z4):
    mesh = plsc.VectorSubcoreMesh(core_axis_name="c", subcore_axis_name="s",
                                  num_cores=NC, num_subcores=NS)
    scratch = [
        pltpu.VMEM((REL_NUM * CW,), jnp.float32),
        pltpu.VMEM((KM * 256,), jnp.int32),
        pltpu.VMEM((KM, 128), jnp.int32),
        pltpu.VMEM((128,), jnp.float32),
        pltpu.VMEM((128,), jnp.float32),
        pltpu.VMEM((128, 8), jnp.float32),
        pltpu.VMEM((128, 8), jnp.float32),
        pltpu.VMEM_SHARED((NPAD, 8), jnp.float32),
        pltpu.SemaphoreType.DMA,
        pltpu.SemaphoreType.DMA,
        pltpu.SemaphoreType.DMA,
        pltpu.SemaphoreType.DMA,
    ]
    f = pl.kernel(_edge_pass2_body,
                  out_type=jax.ShapeDtypeStruct((NC, NPAD, 8),
                                                jnp.float32),
                  mesh=mesh, scratch_types=scratch,
                  compiler_params=pltpu.CompilerParams(
                      needs_layout_passes=False,
                      use_tc_tiling_on_sc=False))
    return f(es, dst, hm, tab1d, z4)


def _mid_body(a_ref, h1_ref, den1_ref):
    a = a_ref[...]
    den0 = a[0, :, 0:4] + a[1, :, 0:4]
    den1 = a[0, :, 4:8] + a[1, :, 4:8]
    s0 = a[0, :, 8:12] + a[1, :, 8:12]
    h1 = s0 / (den0 + 1e-16)
    h1 = jnp.where(h1 > 0, h1, jnp.exp(h1) - 1.0)
    h1_ref[...] = jnp.mean(h1, axis=-1, keepdims=True)
    den1_ref[...] = den1


def _mid(acc):
    grid = (N // _BM,)
    return pl.pallas_call(
        _mid_body,
        grid=grid,
        in_specs=[pl.BlockSpec((2, _BM, CW), lambda i: (0, i, 0))],
        out_specs=[
            pl.BlockSpec((_BM, 1), lambda i: (i, 0)),
            pl.BlockSpec((_BM, HEADS), lambda i: (i, 0)),
        ],
        out_shape=[
            jax.ShapeDtypeStruct((N, 1), jnp.float32),
            jax.ShapeDtypeStruct((N, HEADS), jnp.float32),
        ],
    )(acc)


def _final_body(a_ref, den1_ref, cent_ref, gamma_ref, beta_ref, out_ref):
    a = a_ref[...]
    s1 = a[0, :, 0:4] + a[1, :, 0:4]
    h2 = s1 / (den1_ref[...] + 1e-16)
    h2 = jnp.where(h2 > 0, h2, jnp.exp(h2) - 1.0)
    scaled = (cent_ref[...] * gamma_ref[...] + beta_ref[...]) * h2
    o = jnp.mean(scaled, axis=-1, keepdims=True)
    out_ref[...] = jnp.where(o >= 0, o, 0.01 * o)


def _final(acc, den1, cent, gamma, beta):
    grid = (N // _BM,)
    return pl.pallas_call(
        _final_body,
        grid=grid,
        in_specs=[
            pl.BlockSpec((2, _BM, 8), lambda i: (0, i, 0)),
            pl.BlockSpec((_BM, HEADS), lambda i: (i, 0)),
            pl.BlockSpec((_BM, 1), lambda i: (i, 0)),
            pl.BlockSpec((1, HEADS), lambda i: (0, 0)),
            pl.BlockSpec((1, HEADS), lambda i: (0, 0)),
        ],
        out_specs=pl.BlockSpec((_BM, 1), lambda i: (i, 0)),
        out_shape=jax.ShapeDtypeStruct((N, 1), jnp.float32),
    )(acc, den1, cent, gamma, beta)


def kernel(inputs, W1, b1, W2, b2, rel_emb, Watt, batt, gamma, beta,
           centrality, edge_types, edge_index):
    src = edge_index[0].astype(jnp.int32)
    dst = edge_index[1].astype(jnp.int32).reshape(ROWS, 128)
    et = edge_types.astype(jnp.int32)
    es = jnp.stack([et.reshape(ROWS, 128), src.reshape(ROWS, 128)],
                   axis=1).reshape(ROWS * 256)

    w1cat = W1.transpose(1, 0, 2).reshape(IN_DIM, 2 * IN_DIM)
    b1cat = b1.reshape(1, 2 * IN_DIM)
    w2bd = (jnp.eye(HEADS, dtype=jnp.float32)[:, None, :]
            * W2[:, :, 0][:, :, None]).reshape(2 * IN_DIM, HEADS)
    b2row = b2.reshape(1, HEADS)
    z16 = jnp.zeros((NPAD, CW), jnp.float32)
    z8 = jnp.zeros((NPAD, 8), jnp.float32)

    h0, tab = _mlp(inputs, w1cat, b1cat, w2bd, b2row, rel_emb, Watt, batt)
    tabf = tab.reshape(2, REL_NUM * CW)

    acc1 = _edge_pass(es, dst, h0, tabf[0], z16)
    h1m, den1 = _mid(acc1)
    acc2 = _edge_pass2(es, dst, h1m.reshape(N), tabf[1], z8)
    out = _final(acc2, den1, centrality.reshape(N, 1), gamma, beta)
    return out


# speedup vs baseline: 1.0789x; 1.0789x over previous
import jax
import jax.numpy as jnp
from jax import lax
from jax.experimental import pallas as pl
from jax.experimental.pallas import tpu as pltpu
from jax.experimental.pallas import tpu_sc as plsc

N = 100000
E = 3200000
IN_DIM = 128
HEADS = 4
REL_NUM = 64
NEG = 0.2

LANES = 16
NC = 2
NS = 16
NW = NC * NS
ROWS = E // 128
KM = 25
MACROS = ROWS // KM
NMAC = -(-MACROS // NW)
NPAD = 100096
NT = NPAD // NS
CW = 16
NBUF = 3

_BM = 2000


def _mlp_body(x_ref, w1_ref, b1_ref, w2_ref, b2_ref, rel_ref, watt_ref,
              batt_ref, h_ref, tab_ref):
    x = x_ref[...]
    z = jnp.maximum(
        jnp.dot(x, w1_ref[...], preferred_element_type=jnp.float32)
        + b1_ref[...], 0.0)
    h = (jnp.dot(z, w2_ref[...], preferred_element_type=jnp.float32)
         + b2_ref[...])
    bm = h.shape[0]
    h_ref[...] = jnp.concatenate(
        [jnp.ones((bm, 8), jnp.float32), h, jnp.zeros((bm, 4), jnp.float32)],
        axis=1)

    @pl.when(pl.program_id(0) == 0)
    def _():
        rel = rel_ref[...]
        e0 = (jnp.dot(rel, watt_ref[0], preferred_element_type=jnp.float32)
              + batt_ref[0])
        e1 = (jnp.dot(rel, watt_ref[1], preferred_element_type=jnp.float32)
              + batt_ref[1])
        e0 = jnp.exp(jnp.where(e0 >= 0, e0, NEG * e0))
        e1 = jnp.exp(jnp.where(e1 >= 0, e1, NEG * e1))
        zz = jnp.zeros((REL_NUM, 4), jnp.float32)
        tab_ref[0] = jnp.concatenate([e0, e1, e0, zz], axis=1)
        tab_ref[1] = jnp.concatenate([e0, e1, e1, zz], axis=1)


def _mlp(x, w1cat, b1cat, w2bd, b2row, rel_emb, watt, batt):
    grid = (N // _BM,)
    return pl.pallas_call(
        _mlp_body,
        grid=grid,
        in_specs=[
            pl.BlockSpec((_BM, IN_DIM), lambda i: (i, 0)),
            pl.BlockSpec((IN_DIM, 2 * IN_DIM), lambda i: (0, 0)),
            pl.BlockSpec((1, 2 * IN_DIM), lambda i: (0, 0)),
            pl.BlockSpec((2 * IN_DIM, HEADS), lambda i: (0, 0)),
            pl.BlockSpec((1, HEADS), lambda i: (0, 0)),
            pl.BlockSpec((REL_NUM, 16), lambda i: (0, 0)),
            pl.BlockSpec((2, 16, HEADS), lambda i: (0, 0, 0)),
            pl.BlockSpec((2, HEADS), lambda i: (0, 0)),
        ],
        out_specs=[
            pl.BlockSpec((_BM, CW), lambda i: (i, 0)),
            pl.BlockSpec((2, REL_NUM, CW), lambda i: (0, 0, 0)),
        ],
        out_shape=[
            jax.ShapeDtypeStruct((N, CW), jnp.float32),
            jax.ShapeDtypeStruct((2, REL_NUM, CW), jnp.float32),
        ],
    )(x, w1cat, b1cat, w2bd, b2row, rel_emb, watt, batt)


def _edge_pass_body(es_hbm, dst_hbm, h_hbm, tab_hbm, z_hbm,
                    acc_hbm, tabv, esv, dstv,
                    hrvs, msgxs, acc_sh, gss, sss):
    c = lax.axis_index("c")
    s = lax.axis_index("s")

    pltpu.sync_copy(z_hbm.at[pl.ds(s * NT, NT)], acc_sh.at[pl.ds(s * NT, NT)])
    pltpu.sync_copy(tab_hbm, tabv)
    plsc.subcore_barrier()

    w = c * NS + s
    mstart = (w * MACROS) // NW
    mend = ((w + 1) * MACROS) // NW

    lane = lax.iota(jnp.int32, LANES)
    zeros16 = jnp.full((LANES,), 0, jnp.int32)

    def _gather(jr, buf, sem):
        return pltpu.async_copy(
            h_hbm.at[esv.at[pl.ds(jr * 256 + 128, 128)]], buf, sem)

    def macro(m, carry):
        mm = mstart + m

        @pl.when(mm < mend)
        def _():
            r0 = mm * KM
            pltpu.sync_copy(es_hbm.at[pl.ds(r0 * 256, KM * 256)], esv)
            pltpu.sync_copy(dst_hbm.at[pl.ds(r0, KM)], dstv)

            gd = [None] * KM
            sd = [None] * KM
            for j in range(NBUF):
                gd[j] = _gather(j, hrvs[j], gss[j])
            for j in range(KM):
                b = j % NBUF
                hrv_b, msgx_b = hrvs[b], msgxs[b]
                if j >= NBUF:
                    sd[j - NBUF].wait()
                gd[j].wait()
                base = j * 256

                def edge(e, ce, _hrv=hrv_b, _msgx=msgx_b, _base=base):
                    etb = plsc.load_gather(esv, [zeros16 + (_base + e)])
                    w16 = plsc.load_gather(tabv, [etb * CW + lane])
                    _msgx[e, :] = w16 * _hrv[e, :]
                    return ce
                lax.fori_loop(0, 128, edge, 0, unroll=8)

                sd[j] = pltpu.async_copy(
                    msgx_b, acc_sh.at[dstv.at[j]], sss[b], add=True)
                if j + NBUF < KM:
                    gd[j + NBUF] = _gather(j + NBUF, hrv_b, gss[b])
            for j in range(KM - NBUF, KM):
                sd[j].wait()
        return carry

    lax.fori_loop(0, NMAC, macro, 0)
    plsc.subcore_barrier()

    pltpu.sync_copy(acc_sh.at[pl.ds(s * NT, NT)],
                    acc_hbm.at[c, pl.ds(s * NT, NT)])


def _edge_pass(es, dst, h, tab1d, z16):
    mesh = plsc.VectorSubcoreMesh(core_axis_name="c", subcore_axis_name="s",
                                  num_cores=NC, num_subcores=NS)
    scratch = (
        [pltpu.VMEM((REL_NUM * CW,), jnp.float32),
         pltpu.VMEM((KM * 256,), jnp.int32),
         pltpu.VMEM((KM, 128), jnp.int32)]
        + [pltpu.VMEM((128, CW), jnp.float32)] * NBUF
        + [pltpu.VMEM((128, CW), jnp.float32)] * NBUF
        + [pltpu.VMEM_SHARED((NPAD, CW), jnp.float32)]
        + [pltpu.SemaphoreType.DMA] * (2 * NBUF)
    )

    def body(*refs):
        (es_r, dst_r, h_r, tab_r, z_r, acc_r, tabv, esv, dstv) = refs[:9]
        hrvs = refs[9:9 + NBUF]
        msgxs = refs[9 + NBUF:9 + 2 * NBUF]
        acc_sh = refs[9 + 2 * NBUF]
        gss = refs[10 + 2 * NBUF:10 + 3 * NBUF]
        sss = refs[10 + 3 * NBUF:10 + 4 * NBUF]
        _edge_pass_body(es_r, dst_r, h_r, tab_r, z_r, acc_r,
                        tabv, esv, dstv, hrvs, msgxs, acc_sh, gss, sss)

    f = pl.kernel(body,
                  out_type=jax.ShapeDtypeStruct((NC, NPAD, CW), jnp.float32),
                  mesh=mesh, scratch_types=scratch,
                  compiler_params=pltpu.CompilerParams(
                      needs_layout_passes=False,
                      use_tc_tiling_on_sc=False))
    return f(es, dst, h, tab1d, z16)


def _edge_pass2_body(es_hbm, dst_hbm, hm_hbm, tab_hbm, z_hbm,
                     acc_hbm, tabv, esv, dstv,
                     hms, mss, acc_sh, gss, sss):
    c = lax.axis_index("c")
    s = lax.axis_index("s")

    pltpu.sync_copy(z_hbm.at[pl.ds(s * NT, NT)], acc_sh.at[pl.ds(s * NT, NT)])
    pltpu.sync_copy(tab_hbm, tabv)
    plsc.subcore_barrier()

    w = c * NS + s
    mstart = (w * MACROS) // NW
    mend = ((w + 1) * MACROS) // NW

    lane = lax.iota(jnp.int32, LANES)
    l_d8 = lax.shift_right_logical(lane, 3)
    l_m8 = jnp.bitwise_and(lane, 7)
    l_m8c = jnp.bitwise_and(l_m8, 3)
    m_lo4 = l_m8 < 4
    zeros16 = jnp.full((LANES,), 0, jnp.int32)
    fzeros = jnp.full((LANES,), 0.0, jnp.float32)

    def _gather(jr, buf, sem):
        return pltpu.async_copy(
            hm_hbm.at[esv.at[pl.ds(jr * 256 + 128, 128)]], buf, sem)

    def macro(m, carry):
        mm = mstart + m

        @pl.when(mm < mend)
        def _():
            r0 = mm * KM
            pltpu.sync_copy(es_hbm.at[pl.ds(r0 * 256, KM * 256)], esv)
            pltpu.sync_copy(dst_hbm.at[pl.ds(r0, KM)], dstv)

            gd = [None] * KM
            sd = [None] * KM
            for j in range(NBUF):
                gd[j] = _gather(j, hms[j], gss[j])
            for j in range(KM):
                b = j % NBUF
                hm_b, ms_b = hms[b], mss[b]
                if j >= NBUF:
                    sd[j - NBUF].wait()
                gd[j].wait()
                base = j * 256

                def vec(v, cv, _hm=hm_b, _ms=ms_b, _base=base):
                    e2 = v * 2 + l_d8
                    etg = plsc.load_gather(esv, [zeros16 + _base + e2])
                    wg = plsc.load_gather(tabv, [etg * CW + 8 + l_m8c])
                    hmg = plsc.load_gather(_hm, [e2])
                    val = jnp.where(m_lo4, wg * hmg, fzeros)
                    plsc.store_scatter(_ms, [e2, l_m8], val)
                    return cv
                lax.fori_loop(0, 64, vec, 0, unroll=8)

                sd[j] = pltpu.async_copy(
                    ms_b, acc_sh.at[dstv.at[j]], sss[b], add=True)
                if j + NBUF < KM:
                    gd[j + NBUF] = _gather(j + NBUF, hm_b, gss[b])
            for j in range(KM - NBUF, KM):
                sd[j].wait()
        return carry

    lax.fori_loop(0, NMAC, macro, 0)
    plsc.subcore_barrier()

    pltpu.sync_copy(acc_sh.at[pl.ds(s * NT, NT)],
                    acc_hbm.at[c, pl.ds(s * NT, NT)])


def _edge_pass2(es, dst, hm, tab1d, z8):
    mesh = plsc.VectorSubcoreMesh(core_axis_name="c", subcore_axis_name="s",
                                  num_cores=NC, num_subcores=NS)
    scratch = (
        [pltpu.VMEM((REL_NUM * CW,), jnp.float32),
         pltpu.VMEM((KM * 256,), jnp.int32),
         pltpu.VMEM((KM, 128), jnp.int32)]
        + [pltpu.VMEM((128,), jnp.float32)] * NBUF
        + [pltpu.VMEM((128, 8), jnp.float32)] * NBUF
        + [pltpu.VMEM_SHARED((NPAD, 8), jnp.float32)]
        + [pltpu.SemaphoreType.DMA] * (2 * NBUF)
    )

    def body(*refs):
        (es_r, dst_r, hm_r, tab_r, z_r, acc_r, tabv, esv, dstv) = refs[:9]
        hms = refs[9:9 + NBUF]
        mss = refs[9 + NBUF:9 + 2 * NBUF]
        acc_sh = refs[9 + 2 * NBUF]
        gss = refs[10 + 2 * NBUF:10 + 3 * NBUF]
        sss = refs[10 + 3 * NBUF:10 + 4 * NBUF]
        _edge_pass2_body(es_r, dst_r, hm_r, tab_r, z_r, acc_r,
                         tabv, esv, dstv, hms, mss, acc_sh, gss, sss)

    f = pl.kernel(body,
                  out_type=jax.ShapeDtypeStruct((NC, NPAD, 8),
                                                jnp.float32),
                  mesh=mesh, scratch_types=scratch,
                  compiler_params=pltpu.CompilerParams(
                      needs_layout_passes=False,
                      use_tc_tiling_on_sc=False))
    return f(es, dst, hm, tab1d, z8)


def _mid_body(a_ref, h1_ref, den1_ref):
    a = a_ref[...]
    den0 = a[0, :, 0:4] + a[1, :, 0:4]
    den1 = a[0, :, 4:8] + a[1, :, 4:8]
    s0 = a[0, :, 8:12] + a[1, :, 8:12]
    h1 = s0 / (den0 + 1e-16)
    h1 = jnp.where(h1 > 0, h1, jnp.exp(h1) - 1.0)
    h1_ref[...] = jnp.mean(h1, axis=-1, keepdims=True)
    den1_ref[...] = den1


def _mid(acc):
    grid = (N // _BM,)
    return pl.pallas_call(
        _mid_body,
        grid=grid,
        in_specs=[pl.BlockSpec((2, _BM, CW), lambda i: (0, i, 0))],
        out_specs=[
            pl.BlockSpec((_BM, 1), lambda i: (i, 0)),
            pl.BlockSpec((_BM, HEADS), lambda i: (i, 0)),
        ],
        out_shape=[
            jax.ShapeDtypeStruct((N, 1), jnp.float32),
            jax.ShapeDtypeStruct((N, HEADS), jnp.float32),
        ],
    )(acc)


def _final_body(a_ref, den1_ref, cent_ref, gamma_ref, beta_ref, out_ref):
    a = a_ref[...]
    s1 = a[0, :, 0:4] + a[1, :, 0:4]
    h2 = s1 / (den1_ref[...] + 1e-16)
    h2 = jnp.where(h2 > 0, h2, jnp.exp(h2) - 1.0)
    scaled = (cent_ref[...] * gamma_ref[...] + beta_ref[...]) * h2
    o = jnp.mean(scaled, axis=-1, keepdims=True)
    out_ref[...] = jnp.where(o >= 0, o, 0.01 * o)


def _final(acc, den1, cent, gamma, beta):
    grid = (N // _BM,)
    return pl.pallas_call(
        _final_body,
        grid=grid,
        in_specs=[
            pl.BlockSpec((2, _BM, 8), lambda i: (0, i, 0)),
            pl.BlockSpec((_BM, HEADS), lambda i: (i, 0)),
            pl.BlockSpec((_BM, 1), lambda i: (i, 0)),
            pl.BlockSpec((1, HEADS), lambda i: (0, 0)),
            pl.BlockSpec((1, HEADS), lambda i: (0, 0)),
        ],
        out_specs=pl.BlockSpec((_BM, 1), lambda i: (i, 0)),
        out_shape=jax.ShapeDtypeStruct((N, 1), jnp.float32),
    )(acc, den1, cent, gamma, beta)


def kernel(inputs, W1, b1, W2, b2, rel_emb, Watt, batt, gamma, beta,
           centrality, edge_types, edge_index):
    src = edge_index[0].astype(jnp.int32)
    dst = edge_index[1].astype(jnp.int32).reshape(ROWS, 128)
    et = edge_types.astype(jnp.int32)
    es = jnp.stack([et.reshape(ROWS, 128), src.reshape(ROWS, 128)],
                   axis=1).reshape(ROWS * 256)

    w1cat = W1.transpose(1, 0, 2).reshape(IN_DIM, 2 * IN_DIM)
    b1cat = b1.reshape(1, 2 * IN_DIM)
    w2bd = (jnp.eye(HEADS, dtype=jnp.float32)[:, None, :]
            * W2[:, :, 0][:, :, None]).reshape(2 * IN_DIM, HEADS)
    b2row = b2.reshape(1, HEADS)
    z16 = jnp.zeros((NPAD, CW), jnp.float32)
    z8 = jnp.zeros((NPAD, 8), jnp.float32)

    h0, tab = _mlp(inputs, w1cat, b1cat, w2bd, b2row, rel_emb, Watt, batt)
    tabf = tab.reshape(2, REL_NUM * CW)

    acc1 = _edge_pass(es, dst, h0, tabf[0], z16)
    h1m, den1 = _mid(acc1)
    acc2 = _edge_pass2(es, dst, h1m.reshape(N), tabf[1], z8)
    out = _final(acc2, den1, centrality.reshape(N, 1), gamma, beta)
    return out


# speedup vs baseline: 1.1071x; 1.0262x over previous
import jax
import jax.numpy as jnp
from jax import lax
from jax.experimental import pallas as pl
from jax.experimental.pallas import tpu as pltpu
from jax.experimental.pallas import tpu_sc as plsc

N = 100000
E = 3200000
IN_DIM = 128
HEADS = 4
REL_NUM = 64
NEG = 0.2

LANES = 16
NC = 2
NS = 16
NW = NC * NS
ROWS = E // 128
KM = 20
MACROS = ROWS // KM
NMAC = -(-MACROS // NW)
NPAD = 100096
NT = NPAD // NS
CW = 16
NBUF = 3

_BM = 2000


def _mlp_body(x_ref, w1_ref, b1_ref, w2_ref, b2_ref, rel_ref, watt_ref,
              batt_ref, h_ref, tab_ref):
    x = x_ref[...]
    z = jnp.maximum(
        jnp.dot(x, w1_ref[...], preferred_element_type=jnp.float32)
        + b1_ref[...], 0.0)
    h = (jnp.dot(z, w2_ref[...], preferred_element_type=jnp.float32)
         + b2_ref[...])
    bm = h.shape[0]
    h_ref[...] = jnp.concatenate(
        [jnp.ones((bm, 8), jnp.float32), h, jnp.zeros((bm, 4), jnp.float32)],
        axis=1)

    @pl.when(pl.program_id(0) == 0)
    def _():
        rel = rel_ref[...]
        e0 = (jnp.dot(rel, watt_ref[0], preferred_element_type=jnp.float32)
              + batt_ref[0])
        e1 = (jnp.dot(rel, watt_ref[1], preferred_element_type=jnp.float32)
              + batt_ref[1])
        e0 = jnp.exp(jnp.where(e0 >= 0, e0, NEG * e0))
        e1 = jnp.exp(jnp.where(e1 >= 0, e1, NEG * e1))
        zz = jnp.zeros((REL_NUM, 4), jnp.float32)
        tab_ref[0] = jnp.concatenate([e0, e1, e0, zz], axis=1)
        tab_ref[1] = jnp.concatenate([e0, e1, e1, zz], axis=1)


def _mlp(x, w1cat, b1cat, w2bd, b2row, rel_emb, watt, batt):
    grid = (N // _BM,)
    return pl.pallas_call(
        _mlp_body,
        grid=grid,
        in_specs=[
            pl.BlockSpec((_BM, IN_DIM), lambda i: (i, 0)),
            pl.BlockSpec((IN_DIM, 2 * IN_DIM), lambda i: (0, 0)),
            pl.BlockSpec((1, 2 * IN_DIM), lambda i: (0, 0)),
            pl.BlockSpec((2 * IN_DIM, HEADS), lambda i: (0, 0)),
            pl.BlockSpec((1, HEADS), lambda i: (0, 0)),
            pl.BlockSpec((REL_NUM, 16), lambda i: (0, 0)),
            pl.BlockSpec((2, 16, HEADS), lambda i: (0, 0, 0)),
            pl.BlockSpec((2, HEADS), lambda i: (0, 0)),
        ],
        out_specs=[
            pl.BlockSpec((_BM, CW), lambda i: (i, 0)),
            pl.BlockSpec((2, REL_NUM, CW), lambda i: (0, 0, 0)),
        ],
        out_shape=[
            jax.ShapeDtypeStruct((N, CW), jnp.float32),
            jax.ShapeDtypeStruct((2, REL_NUM, CW), jnp.float32),
        ],
    )(x, w1cat, b1cat, w2bd, b2row, rel_emb, watt, batt)


def _edge_pass_body(es_hbm, dst_hbm, h_hbm, tab_hbm, z_hbm,
                    acc_hbm, tabv, esvs, dstvs,
                    hrvs, msgxs, acc_sh, lss, dss, gss, sss):
    c = lax.axis_index("c")
    s = lax.axis_index("s")

    pltpu.sync_copy(z_hbm.at[pl.ds(s * NT, NT)], acc_sh.at[pl.ds(s * NT, NT)])
    pltpu.sync_copy(tab_hbm, tabv)
    plsc.subcore_barrier()

    w = c * NS + s
    mstart = (w * MACROS) // NW
    mend = ((w + 1) * MACROS) // NW

    lane = lax.iota(jnp.int32, LANES)
    zeros16 = jnp.full((LANES,), 0, jnp.int32)

    def issue_loads(mm, q):
        r0 = mm * KM
        pltpu.async_copy(es_hbm.at[pl.ds(r0 * 256, KM * 256)], esvs[q],
                         lss[q])
        pltpu.async_copy(dst_hbm.at[pl.ds(r0, KM)], dstvs[q], dss[q])

    issue_loads(mstart, 0)

    def macro_pair(p, carry):
        for q in (0, 1):
            mm = mstart + 2 * p + q
            esv, dstv = esvs[q], dstvs[q]

            @pl.when(mm < mend)
            def _(mm=mm, q=q, esv=esv, dstv=dstv):
                pltpu.make_async_copy(
                    es_hbm.at[pl.ds(0, KM * 256)], esv, lss[q]).wait()
                pltpu.make_async_copy(
                    dst_hbm.at[pl.ds(0, KM)], dstv, dss[q]).wait()

                @pl.when(mm + 1 < mend)
                def _issue():
                    issue_loads(mm + 1, 1 - q)

                def _gather(jr, buf, sem):
                    return pltpu.async_copy(
                        h_hbm.at[esv.at[pl.ds(jr * 256 + 128, 128)]],
                        buf, sem)

                gd = [None] * KM
                sd = [None] * KM
                for j in range(NBUF):
                    gd[j] = _gather(j, hrvs[j], gss[j])
                for j in range(KM):
                    b = j % NBUF
                    hrv_b, msgx_b = hrvs[b], msgxs[b]
                    if j >= NBUF:
                        sd[j - NBUF].wait()
                    gd[j].wait()
                    base = j * 256

                    def edge(e, ce, _hrv=hrv_b, _msgx=msgx_b, _base=base):
                        etb = plsc.load_gather(esv, [zeros16 + (_base + e)])
                        w16 = plsc.load_gather(tabv, [etb * CW + lane])
                        _msgx[e, :] = w16 * _hrv[e, :]
                        return ce
                    lax.fori_loop(0, 128, edge, 0, unroll=8)

                    sd[j] = pltpu.async_copy(
                        msgx_b, acc_sh.at[dstv.at[j]], sss[b], add=True)
                    if j + NBUF < KM:
                        gd[j + NBUF] = _gather(j + NBUF, hrv_b, gss[b])
                for j in range(KM - NBUF, KM):
                    sd[j].wait()
        return carry

    lax.fori_loop(0, NMAC // 2, macro_pair, 0)
    plsc.subcore_barrier()

    pltpu.sync_copy(acc_sh.at[pl.ds(s * NT, NT)],
                    acc_hbm.at[c, pl.ds(s * NT, NT)])


def _edge_pass(es, dst, h, tab1d, z16):
    mesh = plsc.VectorSubcoreMesh(core_axis_name="c", subcore_axis_name="s",
                                  num_cores=NC, num_subcores=NS)
    scratch = (
        [pltpu.VMEM((REL_NUM * CW,), jnp.float32)]
        + [pltpu.VMEM((KM * 256,), jnp.int32)] * 2
        + [pltpu.VMEM((KM, 128), jnp.int32)] * 2
        + [pltpu.VMEM((128, CW), jnp.float32)] * NBUF
        + [pltpu.VMEM((128, CW), jnp.float32)] * NBUF
        + [pltpu.VMEM_SHARED((NPAD, CW), jnp.float32)]
        + [pltpu.SemaphoreType.DMA] * (4 + 2 * NBUF)
    )

    def body(*refs):
        (es_r, dst_r, h_r, tab_r, z_r, acc_r, tabv) = refs[:7]
        esvs = refs[7:9]
        dstvs = refs[9:11]
        hrvs = refs[11:11 + NBUF]
        msgxs = refs[11 + NBUF:11 + 2 * NBUF]
        acc_sh = refs[11 + 2 * NBUF]
        k = 12 + 2 * NBUF
        lss = refs[k:k + 2]
        dss = refs[k + 2:k + 4]
        gss = refs[k + 4:k + 4 + NBUF]
        sss = refs[k + 4 + NBUF:k + 4 + 2 * NBUF]
        _edge_pass_body(es_r, dst_r, h_r, tab_r, z_r, acc_r,
                        tabv, esvs, dstvs, hrvs, msgxs, acc_sh,
                        lss, dss, gss, sss)

    f = pl.kernel(body,
                  out_type=jax.ShapeDtypeStruct((NC, NPAD, CW), jnp.float32),
                  mesh=mesh, scratch_types=scratch,
                  compiler_params=pltpu.CompilerParams(
                      needs_layout_passes=False,
                      use_tc_tiling_on_sc=False))
    return f(es, dst, h, tab1d, z16)


def _edge_pass2_body(es_hbm, dst_hbm, hm_hbm, tab_hbm, z_hbm,
                     acc_hbm, tabv, esvs, dstvs,
                     hms, mss, acc_sh, lss, dss, gss, sss):
    c = lax.axis_index("c")
    s = lax.axis_index("s")

    pltpu.sync_copy(z_hbm.at[pl.ds(s * NT, NT)], acc_sh.at[pl.ds(s * NT, NT)])
    pltpu.sync_copy(tab_hbm, tabv)
    plsc.subcore_barrier()

    w = c * NS + s
    mstart = (w * MACROS) // NW
    mend = ((w + 1) * MACROS) // NW

    lane = lax.iota(jnp.int32, LANES)
    l_d8 = lax.shift_right_logical(lane, 3)
    l_m8 = jnp.bitwise_and(lane, 7)
    l_m8c = jnp.bitwise_and(l_m8, 3)
    m_lo4 = l_m8 < 4
    zeros16 = jnp.full((LANES,), 0, jnp.int32)
    fzeros = jnp.full((LANES,), 0.0, jnp.float32)

    def issue_loads(mm, q):
        r0 = mm * KM
        pltpu.async_copy(es_hbm.at[pl.ds(r0 * 256, KM * 256)], esvs[q],
                         lss[q])
        pltpu.async_copy(dst_hbm.at[pl.ds(r0, KM)], dstvs[q], dss[q])

    issue_loads(mstart, 0)

    def macro_pair(p, carry):
        for q in (0, 1):
            mm = mstart + 2 * p + q
            esv, dstv = esvs[q], dstvs[q]

            @pl.when(mm < mend)
            def _(mm=mm, q=q, esv=esv, dstv=dstv):
                pltpu.make_async_copy(
                    es_hbm.at[pl.ds(0, KM * 256)], esv, lss[q]).wait()
                pltpu.make_async_copy(
                    dst_hbm.at[pl.ds(0, KM)], dstv, dss[q]).wait()

                @pl.when(mm + 1 < mend)
                def _issue():
                    issue_loads(mm + 1, 1 - q)

                def _gather(jr, buf, sem):
                    return pltpu.async_copy(
                        hm_hbm.at[esv.at[pl.ds(jr * 256 + 128, 128)]],
                        buf, sem)

                gd = [None] * KM
                sd = [None] * KM
                for j in range(NBUF):
                    gd[j] = _gather(j, hms[j], gss[j])
                for j in range(KM):
                    b = j % NBUF
                    hm_b, ms_b = hms[b], mss[b]
                    if j >= NBUF:
                        sd[j - NBUF].wait()
                    gd[j].wait()
                    base = j * 256

                    def vec(v, cv, _hm=hm_b, _ms=ms_b, _base=base):
                        e2 = v * 2 + l_d8
                        etg = plsc.load_gather(esv, [zeros16 + _base + e2])
                        wg = plsc.load_gather(tabv, [etg * CW + 8 + l_m8c])
                        hmg = plsc.load_gather(_hm, [e2])
                        val = jnp.where(m_lo4, wg * hmg, fzeros)
                        plsc.store_scatter(_ms, [e2, l_m8], val)
                        return cv
                    lax.fori_loop(0, 64, vec, 0, unroll=8)

                    sd[j] = pltpu.async_copy(
                        ms_b, acc_sh.at[dstv.at[j]], sss[b], add=True)
                    if j + NBUF < KM:
                        gd[j + NBUF] = _gather(j + NBUF, hm_b, gss[b])
                for j in range(KM - NBUF, KM):
                    sd[j].wait()
        return carry

    lax.fori_loop(0, NMAC // 2, macro_pair, 0)
    plsc.subcore_barrier()

    pltpu.sync_copy(acc_sh.at[pl.ds(s * NT, NT)],
                    acc_hbm.at[c, pl.ds(s * NT, NT)])


def _edge_pass2(es, dst, hm, tab1d, z8):
    mesh = plsc.VectorSubcoreMesh(core_axis_name="c", subcore_axis_name="s",
                                  num_cores=NC, num_subcores=NS)
    scratch = (
        [pltpu.VMEM((REL_NUM * CW,), jnp.float32)]
        + [pltpu.VMEM((KM * 256,), jnp.int32)] * 2
        + [pltpu.VMEM((KM, 128), jnp.int32)] * 2
        + [pltpu.VMEM((128,), jnp.float32)] * NBUF
        + [pltpu.VMEM((128, 8), jnp.float32)] * NBUF
        + [pltpu.VMEM_SHARED((NPAD, 8), jnp.float32)]
        + [pltpu.SemaphoreType.DMA] * (4 + 2 * NBUF)
    )

    def body(*refs):
        (es_r, dst_r, hm_r, tab_r, z_r, acc_r, tabv) = refs[:7]
        esvs = refs[7:9]
        dstvs = refs[9:11]
        hms = refs[11:11 + NBUF]
        mss = refs[11 + NBUF:11 + 2 * NBUF]
        acc_sh = refs[11 + 2 * NBUF]
        k = 12 + 2 * NBUF
        lss = refs[k:k + 2]
        dss = refs[k + 2:k + 4]
        gss = refs[k + 4:k + 4 + NBUF]
        sss = refs[k + 4 + NBUF:k + 4 + 2 * NBUF]
        _edge_pass2_body(es_r, dst_r, hm_r, tab_r, z_r, acc_r,
                         tabv, esvs, dstvs, hms, mss, acc_sh,
                         lss, dss, gss, sss)

    f = pl.kernel(body,
                  out_type=jax.ShapeDtypeStruct((NC, NPAD, 8),
                                                jnp.float32),
                  mesh=mesh, scratch_types=scratch,
                  compiler_params=pltpu.CompilerParams(
                      needs_layout_passes=False,
                      use_tc_tiling_on_sc=False))
    return f(es, dst, hm, tab1d, z8)


def _mid_body(a_ref, h1_ref, den1_ref):
    a = a_ref[...]
    den0 = a[0, :, 0:4] + a[1, :, 0:4]
    den1 = a[0, :, 4:8] + a[1, :, 4:8]
    s0 = a[0, :, 8:12] + a[1, :, 8:12]
    h1 = s0 / (den0 + 1e-16)
    h1 = jnp.where(h1 > 0, h1, jnp.exp(h1) - 1.0)
    h1_ref[...] = jnp.mean(h1, axis=-1, keepdims=True)
    den1_ref[...] = den1


def _mid(acc):
    grid = (N // _BM,)
    return pl.pallas_call(
        _mid_body,
        grid=grid,
        in_specs=[pl.BlockSpec((2, _BM, CW), lambda i: (0, i, 0))],
        out_specs=[
            pl.BlockSpec((_BM, 1), lambda i: (i, 0)),
            pl.BlockSpec((_BM, HEADS), lambda i: (i, 0)),
        ],
        out_shape=[
            jax.ShapeDtypeStruct((N, 1), jnp.float32),
            jax.ShapeDtypeStruct((N, HEADS), jnp.float32),
        ],
    )(acc)


def _final_body(a_ref, den1_ref, cent_ref, gamma_ref, beta_ref, out_ref):
    a = a_ref[...]
    s1 = a[0, :, 0:4] + a[1, :, 0:4]
    h2 = s1 / (den1_ref[...] + 1e-16)
    h2 = jnp.where(h2 > 0, h2, jnp.exp(h2) - 1.0)
    scaled = (cent_ref[...] * gamma_ref[...] + beta_ref[...]) * h2
    o = jnp.mean(scaled, axis=-1, keepdims=True)
    out_ref[...] = jnp.where(o >= 0, o, 0.01 * o)


def _final(acc, den1, cent, gamma, beta):
    grid = (N // _BM,)
    return pl.pallas_call(
        _final_body,
        grid=grid,
        in_specs=[
            pl.BlockSpec((2, _BM, 8), lambda i: (0, i, 0)),
            pl.BlockSpec((_BM, HEADS), lambda i: (i, 0)),
            pl.BlockSpec((_BM, 1), lambda i: (i, 0)),
            pl.BlockSpec((1, HEADS), lambda i: (0, 0)),
            pl.BlockSpec((1, HEADS), lambda i: (0, 0)),
        ],
        out_specs=pl.BlockSpec((_BM, 1), lambda i: (i, 0)),
        out_shape=jax.ShapeDtypeStruct((N, 1), jnp.float32),
    )(acc, den1, cent, gamma, beta)


def kernel(inputs, W1, b1, W2, b2, rel_emb, Watt, batt, gamma, beta,
           centrality, edge_types, edge_index):
    src = edge_index[0].astype(jnp.int32)
    dst = edge_index[1].astype(jnp.int32).reshape(ROWS, 128)
    et = edge_types.astype(jnp.int32)
    es = jnp.stack([et.reshape(ROWS, 128), src.reshape(ROWS, 128)],
                   axis=1).reshape(ROWS * 256)

    w1cat = W1.transpose(1, 0, 2).reshape(IN_DIM, 2 * IN_DIM)
    b1cat = b1.reshape(1, 2 * IN_DIM)
    w2bd = (jnp.eye(HEADS, dtype=jnp.float32)[:, None, :]
            * W2[:, :, 0][:, :, None]).reshape(2 * IN_DIM, HEADS)
    b2row = b2.reshape(1, HEADS)
    z16 = jnp.zeros((NPAD, CW), jnp.float32)
    z8 = jnp.zeros((NPAD, 8), jnp.float32)

    h0, tab = _mlp(inputs, w1cat, b1cat, w2bd, b2row, rel_emb, Watt, batt)
    tabf = tab.reshape(2, REL_NUM * CW)

    acc1 = _edge_pass(es, dst, h0, tabf[0], z16)
    h1m, den1 = _mid(acc1)
    acc2 = _edge_pass2(es, dst, h1m.reshape(N), tabf[1], z8)
    out = _final(acc2, den1, centrality.reshape(N, 1), gamma, beta)
    return out


# speedup vs baseline: 1.1189x; 1.0106x over previous
import jax
import jax.numpy as jnp
from jax import lax
from jax.experimental import pallas as pl
from jax.experimental.pallas import tpu as pltpu
from jax.experimental.pallas import tpu_sc as plsc

N = 100000
E = 3200000
IN_DIM = 128
HEADS = 4
REL_NUM = 64
NEG = 0.2

LANES = 16
NC = 2
NS = 16
NW = NC * NS
ROWS = E // 128
KM = 20
MACROS = ROWS // KM
NMAC = -(-MACROS // NW)
NPAD = 100096
NT = NPAD // NS
CW = 16
NBUF = 3

_BM = 2000


def _mlp_body(x_ref, w1_ref, b1_ref, w2_ref, b2_ref, rel_ref, watt_ref,
              batt_ref, h_ref, tab_ref):
    x = x_ref[...]
    z = jnp.maximum(
        jnp.dot(x, w1_ref[...], preferred_element_type=jnp.float32)
        + b1_ref[...], 0.0)
    h = (jnp.dot(z, w2_ref[...], preferred_element_type=jnp.float32)
         + b2_ref[...])
    bm = h.shape[0]
    h_ref[...] = jnp.concatenate(
        [jnp.ones((bm, 8), jnp.float32), h, jnp.zeros((bm, 4), jnp.float32)],
        axis=1)

    @pl.when(pl.program_id(0) == 0)
    def _():
        rel = rel_ref[...]
        e0 = (jnp.dot(rel, watt_ref[0], preferred_element_type=jnp.float32)
              + batt_ref[0])
        e1 = (jnp.dot(rel, watt_ref[1], preferred_element_type=jnp.float32)
              + batt_ref[1])
        e0 = jnp.exp(jnp.where(e0 >= 0, e0, NEG * e0))
        e1 = jnp.exp(jnp.where(e1 >= 0, e1, NEG * e1))
        zz = jnp.zeros((REL_NUM, 4), jnp.float32)
        tab_ref[0] = jnp.concatenate([e0, e1, e0, zz], axis=1)
        tab_ref[1] = jnp.concatenate([e0, e1, e1, zz], axis=1)


def _mlp(x, w1cat, b1cat, w2bd, b2row, rel_emb, watt, batt):
    grid = (N // _BM,)
    return pl.pallas_call(
        _mlp_body,
        grid=grid,
        in_specs=[
            pl.BlockSpec((_BM, IN_DIM), lambda i: (i, 0)),
            pl.BlockSpec((IN_DIM, 2 * IN_DIM), lambda i: (0, 0)),
            pl.BlockSpec((1, 2 * IN_DIM), lambda i: (0, 0)),
            pl.BlockSpec((2 * IN_DIM, HEADS), lambda i: (0, 0)),
            pl.BlockSpec((1, HEADS), lambda i: (0, 0)),
            pl.BlockSpec((REL_NUM, 16), lambda i: (0, 0)),
            pl.BlockSpec((2, 16, HEADS), lambda i: (0, 0, 0)),
            pl.BlockSpec((2, HEADS), lambda i: (0, 0)),
        ],
        out_specs=[
            pl.BlockSpec((_BM, CW), lambda i: (i, 0)),
            pl.BlockSpec((2, REL_NUM, CW), lambda i: (0, 0, 0)),
        ],
        out_shape=[
            jax.ShapeDtypeStruct((N, CW), jnp.float32),
            jax.ShapeDtypeStruct((2, REL_NUM, CW), jnp.float32),
        ],
    )(x, w1cat, b1cat, w2bd, b2row, rel_emb, watt, batt)


def _edge_pass_body(es_hbm, dst_hbm, h_hbm, tab_hbm, z_hbm,
                    acc_hbm, tabv, esvs, dstvs,
                    hrvs, msgxs, acc_sh, lss, dss, gss, sss):
    c = lax.axis_index("c")
    s = lax.axis_index("s")

    pltpu.sync_copy(z_hbm, acc_sh.at[pl.ds(s * NT, NT)])
    pltpu.sync_copy(tab_hbm, tabv)
    plsc.subcore_barrier()

    w = c * NS + s
    mstart = (w * MACROS) // NW
    mend = ((w + 1) * MACROS) // NW

    lane = lax.iota(jnp.int32, LANES)
    zeros16 = jnp.full((LANES,), 0, jnp.int32)

    def issue_loads(mm, q):
        r0 = mm * KM
        pltpu.async_copy(es_hbm.at[pl.ds(r0 * 256, KM * 256)], esvs[q],
                         lss[q])
        pltpu.async_copy(dst_hbm.at[pl.ds(r0, KM)], dstvs[q], dss[q])

    issue_loads(mstart, 0)

    def macro_pair(p, carry):
        for q in (0, 1):
            mm = mstart + 2 * p + q
            esv, dstv = esvs[q], dstvs[q]

            @pl.when(mm < mend)
            def _(mm=mm, q=q, esv=esv, dstv=dstv):
                pltpu.make_async_copy(
                    es_hbm.at[pl.ds(0, KM * 256)], esv, lss[q]).wait()
                pltpu.make_async_copy(
                    dst_hbm.at[pl.ds(0, KM)], dstv, dss[q]).wait()

                @pl.when(mm + 1 < mend)
                def _issue():
                    issue_loads(mm + 1, 1 - q)

                def _gather(jr, buf, sem):
                    return pltpu.async_copy(
                        h_hbm.at[esv.at[pl.ds(jr * 256 + 128, 128)]],
                        buf, sem)

                gd = [None] * KM
                sd = [None] * KM
                for j in range(NBUF):
                    gd[j] = _gather(j, hrvs[j], gss[j])
                for j in range(KM):
                    b = j % NBUF
                    hrv_b, msgx_b = hrvs[b], msgxs[b]
                    if j >= NBUF:
                        sd[j - NBUF].wait()
                    gd[j].wait()
                    base = j * 256

                    def edge(e, ce, _hrv=hrv_b, _msgx=msgx_b, _base=base):
                        etb = plsc.load_gather(esv, [zeros16 + (_base + e)])
                        w16 = plsc.load_gather(tabv, [etb * CW + lane])
                        _msgx[e, :] = w16 * _hrv[e, :]
                        return ce
                    lax.fori_loop(0, 128, edge, 0, unroll=16)

                    sd[j] = pltpu.async_copy(
                        msgx_b, acc_sh.at[dstv.at[j]], sss[b], add=True)
                    if j + NBUF < KM:
                        gd[j + NBUF] = _gather(j + NBUF, hrv_b, gss[b])
                for j in range(KM - NBUF, KM):
                    sd[j].wait()
        return carry

    lax.fori_loop(0, NMAC // 2, macro_pair, 0)
    plsc.subcore_barrier()

    pltpu.sync_copy(acc_sh.at[pl.ds(s * NT, NT)],
                    acc_hbm.at[c, pl.ds(s * NT, NT)])


def _edge_pass(es, dst, h, tab1d, z16):
    mesh = plsc.VectorSubcoreMesh(core_axis_name="c", subcore_axis_name="s",
                                  num_cores=NC, num_subcores=NS)
    scratch = (
        [pltpu.VMEM((REL_NUM * CW,), jnp.float32)]
        + [pltpu.VMEM((KM * 256,), jnp.int32)] * 2
        + [pltpu.VMEM((KM, 128), jnp.int32)] * 2
        + [pltpu.VMEM((128, CW), jnp.float32)] * NBUF
        + [pltpu.VMEM((128, CW), jnp.float32)] * NBUF
        + [pltpu.VMEM_SHARED((NPAD, CW), jnp.float32)]
        + [pltpu.SemaphoreType.DMA] * (4 + 2 * NBUF)
    )

    def body(*refs):
        (es_r, dst_r, h_r, tab_r, z_r, acc_r, tabv) = refs[:7]
        esvs = refs[7:9]
        dstvs = refs[9:11]
        hrvs = refs[11:11 + NBUF]
        msgxs = refs[11 + NBUF:11 + 2 * NBUF]
        acc_sh = refs[11 + 2 * NBUF]
        k = 12 + 2 * NBUF
        lss = refs[k:k + 2]
        dss = refs[k + 2:k + 4]
        gss = refs[k + 4:k + 4 + NBUF]
        sss = refs[k + 4 + NBUF:k + 4 + 2 * NBUF]
        _edge_pass_body(es_r, dst_r, h_r, tab_r, z_r, acc_r,
                        tabv, esvs, dstvs, hrvs, msgxs, acc_sh,
                        lss, dss, gss, sss)

    f = pl.kernel(body,
                  out_type=jax.ShapeDtypeStruct((NC, NPAD, CW), jnp.float32),
                  mesh=mesh, scratch_types=scratch,
                  compiler_params=pltpu.CompilerParams(
                      needs_layout_passes=False,
                      use_tc_tiling_on_sc=False))
    return f(es, dst, h, tab1d, z16)


def _edge_pass2_body(es_hbm, dst_hbm, hm_hbm, tab_hbm, z_hbm,
                     acc_hbm, tabv, esvs, dstvs,
                     hms, mss, acc_sh, lss, dss, gss, sss):
    c = lax.axis_index("c")
    s = lax.axis_index("s")

    pltpu.sync_copy(z_hbm, acc_sh.at[pl.ds(s * NT, NT)])
    pltpu.sync_copy(tab_hbm, tabv)
    plsc.subcore_barrier()

    w = c * NS + s
    mstart = (w * MACROS) // NW
    mend = ((w + 1) * MACROS) // NW

    lane = lax.iota(jnp.int32, LANES)
    l_d8 = lax.shift_right_logical(lane, 3)
    l_m8 = jnp.bitwise_and(lane, 7)
    l_m8c = jnp.bitwise_and(l_m8, 3)
    m_lo4 = l_m8 < 4
    zeros16 = jnp.full((LANES,), 0, jnp.int32)
    fzeros = jnp.full((LANES,), 0.0, jnp.float32)

    def issue_loads(mm, q):
        r0 = mm * KM
        pltpu.async_copy(es_hbm.at[pl.ds(r0 * 256, KM * 256)], esvs[q],
                         lss[q])
        pltpu.async_copy(dst_hbm.at[pl.ds(r0, KM)], dstvs[q], dss[q])

    issue_loads(mstart, 0)

    def macro_pair(p, carry):
        for q in (0, 1):
            mm = mstart + 2 * p + q
            esv, dstv = esvs[q], dstvs[q]

            @pl.when(mm < mend)
            def _(mm=mm, q=q, esv=esv, dstv=dstv):
                pltpu.make_async_copy(
                    es_hbm.at[pl.ds(0, KM * 256)], esv, lss[q]).wait()
                pltpu.make_async_copy(
                    dst_hbm.at[pl.ds(0, KM)], dstv, dss[q]).wait()

                @pl.when(mm + 1 < mend)
                def _issue():
                    issue_loads(mm + 1, 1 - q)

                def _gather(jr, buf, sem):
                    return pltpu.async_copy(
                        hm_hbm.at[esv.at[pl.ds(jr * 256 + 128, 128)]],
                        buf, sem)

                gd = [None] * KM
                sd = [None] * KM
                for j in range(NBUF):
                    gd[j] = _gather(j, hms[j], gss[j])
                for j in range(KM):
                    b = j % NBUF
                    hm_b, ms_b = hms[b], mss[b]
                    if j >= NBUF:
                        sd[j - NBUF].wait()
                    gd[j].wait()
                    base = j * 256

                    def vec(v, cv, _hm=hm_b, _ms=ms_b, _base=base):
                        e2 = v * 2 + l_d8
                        etg = plsc.load_gather(esv, [zeros16 + _base + e2])
                        wg = plsc.load_gather(tabv, [etg * CW + 8 + l_m8c])
                        hmg = plsc.load_gather(_hm, [e2])
                        val = jnp.where(m_lo4, wg * hmg, fzeros)
                        plsc.store_scatter(_ms, [e2, l_m8], val)
                        return cv
                    lax.fori_loop(0, 64, vec, 0, unroll=8)

                    sd[j] = pltpu.async_copy(
                        ms_b, acc_sh.at[dstv.at[j]], sss[b], add=True)
                    if j + NBUF < KM:
                        gd[j + NBUF] = _gather(j + NBUF, hm_b, gss[b])
                for j in range(KM - NBUF, KM):
                    sd[j].wait()
        return carry

    lax.fori_loop(0, NMAC // 2, macro_pair, 0)
    plsc.subcore_barrier()

    pltpu.sync_copy(acc_sh.at[pl.ds(s * NT, NT)],
                    acc_hbm.at[c, pl.ds(s * NT, NT)])


def _edge_pass2(es, dst, hm, tab1d, z8):
    mesh = plsc.VectorSubcoreMesh(core_axis_name="c", subcore_axis_name="s",
                                  num_cores=NC, num_subcores=NS)
    scratch = (
        [pltpu.VMEM((REL_NUM * CW,), jnp.float32)]
        + [pltpu.VMEM((KM * 256,), jnp.int32)] * 2
        + [pltpu.VMEM((KM, 128), jnp.int32)] * 2
        + [pltpu.VMEM((128,), jnp.float32)] * NBUF
        + [pltpu.VMEM((128, 8), jnp.float32)] * NBUF
        + [pltpu.VMEM_SHARED((NPAD, 8), jnp.float32)]
        + [pltpu.SemaphoreType.DMA] * (4 + 2 * NBUF)
    )

    def body(*refs):
        (es_r, dst_r, hm_r, tab_r, z_r, acc_r, tabv) = refs[:7]
        esvs = refs[7:9]
        dstvs = refs[9:11]
        hms = refs[11:11 + NBUF]
        mss = refs[11 + NBUF:11 + 2 * NBUF]
        acc_sh = refs[11 + 2 * NBUF]
        k = 12 + 2 * NBUF
        lss = refs[k:k + 2]
        dss = refs[k + 2:k + 4]
        gss = refs[k + 4:k + 4 + NBUF]
        sss = refs[k + 4 + NBUF:k + 4 + 2 * NBUF]
        _edge_pass2_body(es_r, dst_r, hm_r, tab_r, z_r, acc_r,
                         tabv, esvs, dstvs, hms, mss, acc_sh,
                         lss, dss, gss, sss)

    f = pl.kernel(body,
                  out_type=jax.ShapeDtypeStruct((NC, NPAD, 8),
                                                jnp.float32),
                  mesh=mesh, scratch_types=scratch,
                  compiler_params=pltpu.CompilerParams(
                      needs_layout_passes=False,
                      use_tc_tiling_on_sc=False))
    return f(es, dst, hm, tab1d, z8)


def _mid_body(a_ref, h1_ref, den1_ref):
    a = a_ref[...]
    den0 = a[0, :, 0:4] + a[1, :, 0:4]
    den1 = a[0, :, 4:8] + a[1, :, 4:8]
    s0 = a[0, :, 8:12] + a[1, :, 8:12]
    h1 = s0 / (den0 + 1e-16)
    h1 = jnp.where(h1 > 0, h1, jnp.exp(h1) - 1.0)
    h1_ref[...] = jnp.mean(h1, axis=-1, keepdims=True)
    den1_ref[...] = den1


def _mid(acc):
    grid = (N // _BM,)
    return pl.pallas_call(
        _mid_body,
        grid=grid,
        in_specs=[pl.BlockSpec((2, _BM, CW), lambda i: (0, i, 0))],
        out_specs=[
            pl.BlockSpec((_BM, 1), lambda i: (i, 0)),
            pl.BlockSpec((_BM, HEADS), lambda i: (i, 0)),
        ],
        out_shape=[
            jax.ShapeDtypeStruct((N, 1), jnp.float32),
            jax.ShapeDtypeStruct((N, HEADS), jnp.float32),
        ],
    )(acc)


def _final_body(a_ref, den1_ref, cent_ref, gamma_ref, beta_ref, out_ref):
    a = a_ref[...]
    s1 = a[0, :, 0:4] + a[1, :, 0:4]
    h2 = s1 / (den1_ref[...] + 1e-16)
    h2 = jnp.where(h2 > 0, h2, jnp.exp(h2) - 1.0)
    scaled = (cent_ref[...] * gamma_ref[...] + beta_ref[...]) * h2
    o = jnp.mean(scaled, axis=-1, keepdims=True)
    out_ref[...] = jnp.where(o >= 0, o, 0.01 * o)


def _final(acc, den1, cent, gamma, beta):
    grid = (N // _BM,)
    return pl.pallas_call(
        _final_body,
        grid=grid,
        in_specs=[
            pl.BlockSpec((2, _BM, 8), lambda i: (0, i, 0)),
            pl.BlockSpec((_BM, HEADS), lambda i: (i, 0)),
            pl.BlockSpec((_BM, 1), lambda i: (i, 0)),
            pl.BlockSpec((1, HEADS), lambda i: (0, 0)),
            pl.BlockSpec((1, HEADS), lambda i: (0, 0)),
        ],
        out_specs=pl.BlockSpec((_BM, 1), lambda i: (i, 0)),
        out_shape=jax.ShapeDtypeStruct((N, 1), jnp.float32),
    )(acc, den1, cent, gamma, beta)


def kernel(inputs, W1, b1, W2, b2, rel_emb, Watt, batt, gamma, beta,
           centrality, edge_types, edge_index):
    src = edge_index[0].astype(jnp.int32)
    dst = edge_index[1].astype(jnp.int32).reshape(ROWS, 128)
    et = edge_types.astype(jnp.int32)
    es = jnp.stack([et.reshape(ROWS, 128), src.reshape(ROWS, 128)],
                   axis=1).reshape(ROWS * 256)

    w1cat = W1.transpose(1, 0, 2).reshape(IN_DIM, 2 * IN_DIM)
    b1cat = b1.reshape(1, 2 * IN_DIM)
    w2bd = (jnp.eye(HEADS, dtype=jnp.float32)[:, None, :]
            * W2[:, :, 0][:, :, None]).reshape(2 * IN_DIM, HEADS)
    b2row = b2.reshape(1, HEADS)
    z16 = jnp.zeros((NT, CW), jnp.float32)
    z8 = jnp.zeros((NT, 8), jnp.float32)

    h0, tab = _mlp(inputs, w1cat, b1cat, w2bd, b2row, rel_emb, Watt, batt)
    tabf = tab.reshape(2, REL_NUM * CW)

    acc1 = _edge_pass(es, dst, h0, tabf[0], z16)
    h1m, den1 = _mid(acc1)
    acc2 = _edge_pass2(es, dst, h1m.reshape(N), tabf[1], z8)
    out = _final(acc2, den1, centrality.reshape(N, 1), gamma, beta)
    return out
